# Initial kernel scaffold; baseline (speedup 1.0000x reference)
import functools
import jax
import jax.numpy as jnp
from jax import lax
from jax.experimental import pallas as pl
from jax.experimental.pallas import tpu as pltpu
from jax.experimental.pallas import tpu_sc as plsc

NC, NS = 2, 16
NW = NC * NS


def kernel(inp_positions, out_positions):
    N = inp_positions.shape[0]
    M = out_positions.shape[0]
    inp_flat = inp_positions.reshape(-1)

    mesh = plsc.VectorSubcoreMesh(
        core_axis_name="c", subcore_axis_name="s", num_cores=NC, num_subcores=NS)

    @functools.partial(
        pl.kernel,
        out_type=[
            jax.ShapeDtypeStruct((M, 64), jnp.int32),
            jax.ShapeDtypeStruct((M, 64), jnp.float32),
        ],
        mesh=mesh,
        scratch_types=[
            pltpu.VMEM((4096,), jnp.float32),
            pltpu.VMEM((1024,), jnp.float32),
            pltpu.VMEM((1024,), jnp.int32),
            pltpu.VMEM((16 * 64,), jnp.int32),
            pltpu.VMEM((16 * 64,), jnp.float32),
            pltpu.VMEM((256,), jnp.int32),
        ],
    )
    def probe(inp_hbm, out_hbm, oidx_hbm, odist_hbm,
              pos_v, key_v, val_v, oi_v, od_v, hist_v):
        wid = lax.axis_index("s") * NC + lax.axis_index("c")
        pltpu.sync_copy(inp_hbm.at[pl.ds(0, 4096)], pos_v)
        iota = lax.iota(jnp.int32, 16)

        def body(i, carry):
            off, acc = carry
            x = pos_v[pl.ds(i * 16, 16)]
            g = plsc.load_gather(pos_v, [(i * 3 + 3 * iota) % 4096])
            d2 = x * x + g
            keep = d2 <= 0.5
            plsc.store_compressed(key_v.at[pl.ds(off, 16)], d2, mask=keep)
            plsc.store_compressed(val_v.at[pl.ds(off, 16)], iota + i, mask=keep)
            cnt = jnp.sum(keep.astype(jnp.int32))
            hist_v[i] = cnt
            prev = hist_v[(i + 1) % 256]
            return off + cnt, acc + prev

        off, acc = lax.fori_loop(0, 64, body, (jnp.int32(0), jnp.int32(0)))

        k0 = key_v[pl.ds(0, 16)]
        v0 = val_v[pl.ds(0, 16)]
        k1 = key_v[pl.ds(16, 16)]
        v1 = val_v[pl.ds(16, 16)]
        k0, v0 = plsc.sort_key_val(k0, v0)
        k1, v1 = plsc.sort_key_val(k1, v1)
        rk1 = lax.rev(k1, (0,))
        rv1 = lax.rev(v1, (0,))
        m = k0 <= rk1
        lo_k = jnp.where(m, k0, rk1)
        lo_v = jnp.where(m, v0, rv1)
        hi_k = jnp.where(m, rk1, k0)
        lo_k, lo_v = plsc.sort_key_val(lo_k, lo_v)

        def small(_):
            return lo_k
        def big(_):
            return hi_k
        kk = lax.cond(off <= 16, small, big, None)

        qx = pos_v[3 * (off % 64)]
        qv = jnp.full((16,), qx, jnp.float32) * kk

        od_v[pl.ds(0, 16)] = qv
        oi_v[pl.ds(0, 16)] = jnp.where(m, lo_v, v0 + acc)

        qlo = wid * 16
        @pl.when(wid < 31)
        def _():
            pltpu.sync_copy(oi_v, oidx_hbm.at[pl.ds(qlo, 16)].reshape(16 * 64))
            pltpu.sync_copy(od_v, odist_hbm.at[pl.ds(qlo, 16)].reshape(16 * 64))

    oi, od = probe(inp_flat, out_positions.reshape(-1))
    counts = jnp.sum(oi >= 0, axis=1).astype(jnp.int32)
    row_splits = jnp.concatenate(
        [jnp.zeros((1,), jnp.int32), jnp.cumsum(counts).astype(jnp.int32)])
    return oi, row_splits, od


# speedup vs baseline: 50.4332x; 50.4332x over previous
import functools
import numpy as np
import jax
import jax.numpy as jnp
from jax import lax
from jax.experimental import pallas as pl
from jax.experimental.pallas import tpu as pltpu
from jax.experimental.pallas import tpu_sc as plsc

NC, NS = 2, 16
NW = NC * NS
G = 20
NCELL = G * G * G
SLABC = 256
CPAD = NW * SLABC
SCAP = 1024
PCAP = 1536
KCAP = 1024
R2F = float(np.float32(0.05 * 0.05))
SENT = 1 << 20


def _mesh():
    return plsc.VectorSubcoreMesh(
        core_axis_name="c", subcore_axis_name="s",
        num_cores=NC, num_subcores=NS)


def _wid():
    return lax.axis_index("s") * NC + lax.axis_index("c")


def _iota():
    return lax.iota(jnp.int32, 16)


def _sortv(k, v):
    return plsc.sort_key_val(k, v)


def _merge16(ak, av, bk, bv):
    rk = lax.rev(bk, (0,))
    rv = lax.rev(bv, (0,))
    m = ak <= rk
    lk = jnp.where(m, ak, rk)
    lv = jnp.where(m, av, rv)
    hk = jnp.where(m, rk, ak)
    hv = jnp.where(m, rv, av)
    lk, lv = _sortv(lk, lv)
    hk, hv = _sortv(hk, hv)
    return lk, lv, hk, hv


def _ce(ak, av, bk, bv):
    m = ak <= bk
    return (jnp.where(m, ak, bk), jnp.where(m, av, bv),
            jnp.where(m, bk, ak), jnp.where(m, bv, av))


def _sort64(k0, v0, k1, v1, k2, v2, k3, v3):
    k0, v0 = _sortv(k0, v0)
    k1, v1 = _sortv(k1, v1)
    k2, v2 = _sortv(k2, v2)
    k3, v3 = _sortv(k3, v3)
    a0k, a0v, a1k, a1v = _merge16(k0, v0, k1, v1)
    b0k, b0v, b1k, b1v = _merge16(k2, v2, k3, v3)
    l0k, l0v, h0k, h0v = _ce(a0k, a0v, lax.rev(b1k, (0,)), lax.rev(b1v, (0,)))
    l1k, l1v, h1k, h1v = _ce(a1k, a1v, lax.rev(b0k, (0,)), lax.rev(b0v, (0,)))
    r0k, r0v, r1k, r1v = _ce(l0k, l0v, l1k, l1v)
    r2k, r2v, r3k, r3v = _ce(h0k, h0v, h1k, h1v)
    r0k, r0v = _sortv(r0k, r0v)
    r1k, r1v = _sortv(r1k, r1v)
    r2k, r2v = _sortv(r2k, r2v)
    r3k, r3v = _sortv(r3k, r3v)
    return r0k, r0v, r1k, r1v, r2k, r2v, r3k, r3v


def _chunk_ranks(lc, nvalid):
    iota = _iota()
    valid0 = iota < nvalid
    lcs, perm = _sortv(jnp.where(valid0, lc, SENT), iota)
    valid = iota < nvalid
    prv = jnp.take(lcs, jnp.maximum(iota - 1, 0))
    nxt = jnp.take(lcs, jnp.minimum(iota + 1, 15))
    first = (lcs != prv) | (iota == 0)
    last = (lcs != nxt) | (iota == 15)
    rank = iota - plsc.cummax(jnp.where(first, iota, 0))
    return lcs, perm, rank, last, valid


def _tie_fix(R, parity):
    iota = _iota()
    iop = jnp.minimum(iota + 1, 15)
    iom = jnp.maximum(iota - 1, 0)
    left = (iota % 2) == parity
    ks = [R[0], R[2], R[4], R[6]]
    vs = [R[1], R[3], R[5], R[7]]
    out = []
    for t in range(4):
        k, v = ks[t], vs[t]
        nk = jnp.take(k, iop)
        nv = jnp.take(v, iop)
        if t < 3:
            is15 = iota == 15
            nk = jnp.where(is15, jnp.full((16,), ks[t + 1][0], jnp.float32), nk)
            nv = jnp.where(is15, jnp.full((16,), vs[t + 1][0], jnp.int32), nv)
            nextok = jnp.full((16,), True)
        else:
            nextok = iota != 15
        pk = jnp.take(k, iom)
        pv = jnp.take(v, iom)
        if t > 0:
            is0 = iota == 0
            pk = jnp.where(is0, jnp.full((16,), ks[t - 1][15], jnp.float32), pk)
            pv = jnp.where(is0, jnp.full((16,), vs[t - 1][15], jnp.int32), pv)
            prevok = jnp.full((16,), True)
        else:
            prevok = iota != 0
        swap_l = left & nextok & (k == nk) & (v > nv)
        swap_r = (~left) & prevok & (k == pk) & (pv > v)
        out.extend([k, jnp.where(swap_l, nv, jnp.where(swap_r, pv, v))])
    return tuple(out)


def _bf16_rne(v):
    u = plsc.bitcast(v, jnp.int32)
    r = jnp.bitwise_and(u + 0x7FFF + jnp.bitwise_and(u >> 16, 1), -65536)
    return plsc.bitcast(r, jnp.float32)


def _exact_sum3(p0, p1, p2):
    one = jnp.int32(1)
    us = [plsc.bitcast(p, jnp.int32) for p in (p0, p1, p2)]
    es = [u >> 23 for u in us]
    eeff = [jnp.maximum(e, 1) for e in es]
    ms = [jnp.bitwise_and(u, 0x7FFFFF)
          | jnp.where(e > 0, jnp.int32(0x800000), jnp.int32(0))
          for u, e in zip(us, es)]
    emax = jnp.maximum(jnp.maximum(eeff[0], eeff[1]), eeff[2])
    s = jnp.zeros((16,), jnp.int32)
    st = jnp.zeros((16,), jnp.bool_)
    for m, e in zip(ms, eeff):
        d = jnp.minimum(emax - e, 31)
        m5 = m << 5
        s = s + (m5 >> d)
        st = st | (jnp.bitwise_and(m5, (one << d) - 1) != 0)
    r = (5 + (s >= (1 << 29)).astype(jnp.int32)
         + (s >= (1 << 30)).astype(jnp.int32))
    low = jnp.bitwise_and(s, (one << r) - 1)
    m_out = s >> r
    rb = jnp.bitwise_and(low >> (r - 1), 1)
    st2 = st | (jnp.bitwise_and(low, (one << (r - 1)) - 1) != 0)
    odd = jnp.bitwise_and(m_out, 1) == 1
    m_out = m_out + rb * (st2 | odd).astype(jnp.int32)
    bout = emax + (r - 5)
    ovf = m_out >= (1 << 24)
    m_out = jnp.where(ovf, m_out >> 1, m_out)
    bout = jnp.where(ovf, bout + 1, bout)
    norm = m_out >= (1 << 23)
    u_out = jnp.where(norm,
                      (bout << 23) | jnp.bitwise_and(m_out, 0x7FFFFF),
                      m_out)
    return plsc.bitcast(u_out, jnp.float32)


def _isqrt(d):
    y = plsc.bitcast(jnp.int32(0x5F3759DF) - (plsc.bitcast(d, jnp.int32) >> 1),
                     jnp.float32)
    for _ in range(3):
        y = y * (1.5 - 0.5 * d * y * y)
    return y


def kernel(inp_positions, out_positions):
    N, _ = inp_positions.shape
    M, _ = out_positions.shape
    Np = -(-N // 16) * 16
    QPW = ((-(-M // NW)) + 15) // 16 * 16
    nchunks = Np // 16
    stage_cap = Np + 8 * NW + SCAP + 16

    inp_flat = inp_positions.reshape(-1)
    if Np != N:
        inp_flat = jnp.concatenate(
            [inp_flat, jnp.full(((Np - N) * 3,), 2.0, jnp.float32)])
    q_flat = out_positions.reshape(-1)
    qpad = NW * QPW * 3
    if qpad != q_flat.shape[0]:
        q_flat = jnp.concatenate(
            [q_flat, jnp.full((qpad - q_flat.shape[0],), 2.0, jnp.float32)])

    mesh = _mesh()
    cparams = pltpu.CompilerParams(needs_layout_passes=False)

    @functools.partial(
        pl.kernel,
        out_type=[
            jax.ShapeDtypeStruct((NW * SCAP,), jnp.float32),
            jax.ShapeDtypeStruct((NW * SCAP,), jnp.float32),
            jax.ShapeDtypeStruct((NW * SCAP,), jnp.float32),
            jax.ShapeDtypeStruct((NW * SCAP,), jnp.int32),
            jax.ShapeDtypeStruct((CPAD,), jnp.int32),
            jax.ShapeDtypeStruct((NW * 8,), jnp.int32),
        ],
        mesh=mesh,
        compiler_params=cparams,
        scratch_types=[
            pltpu.VMEM((Np * 3,), jnp.float32),
            pltpu.VMEM((PCAP + 16,), jnp.int32),
            pltpu.VMEM((PCAP + 16,), jnp.int32),
            pltpu.VMEM((PCAP + 16,), jnp.float32),
            pltpu.VMEM((PCAP + 16,), jnp.float32),
            pltpu.VMEM((PCAP + 16,), jnp.float32),
            pltpu.VMEM((SLABC + 16,), jnp.int32),
            pltpu.VMEM((SLABC + 16,), jnp.int32),
            pltpu.VMEM((SLABC + 16,), jnp.int32),
            pltpu.VMEM((SCAP + 16,), jnp.float32),
            pltpu.VMEM((SCAP + 16,), jnp.float32),
            pltpu.VMEM((SCAP + 16,), jnp.float32),
            pltpu.VMEM((SCAP + 16,), jnp.int32),
            pltpu.VMEM((16,), jnp.int32),
        ],
    )
    def bin_kernel(inp_hbm, sx_hbm, sy_hbm, sz_hbm, si_hbm, csl_hbm, stt_hbm,
                   pos_v, ccid_v, cidx_v, cx_v, cy_v, cz_v,
                   hist_v, starts_v, cursor_v, bx_v, by_v, bz_v, bi_v, st_v):
        wid = _wid()
        iota = _iota()
        pltpu.sync_copy(inp_hbm, pos_v)
        cell_lo = wid * SLABC
        zero = jnp.zeros((16,), jnp.int32)

        def pass1(i, carry):
            off, nbelow = carry
            base3 = i * 48
            gx = plsc.load_gather(pos_v, [base3 + 3 * iota])
            gy = plsc.load_gather(pos_v, [base3 + 3 * iota + 1])
            gz = plsc.load_gather(pos_v, [base3 + 3 * iota + 2])
            ix = jnp.clip((gx * 20.0).astype(jnp.int32), 0, G - 1)
            iy = jnp.clip((gy * 20.0).astype(jnp.int32), 0, G - 1)
            iz = jnp.clip((gz * 20.0).astype(jnp.int32), 0, G - 1)
            cid = (ix * G + iy) * G + iz
            own = (cid >= cell_lo) & (cid < cell_lo + SLABC)
            below = cid < cell_lo
            offc = jnp.minimum(off, PCAP)
            sl = pl.ds(offc, 16)
            plsc.store_compressed(ccid_v.at[sl], cid - cell_lo, mask=own)
            plsc.store_compressed(cidx_v.at[sl], i * 16 + iota, mask=own)
            plsc.store_compressed(cx_v.at[sl], gx, mask=own)
            plsc.store_compressed(cy_v.at[sl], gy, mask=own)
            plsc.store_compressed(cz_v.at[sl], gz, mask=own)
            return (off + jnp.sum(own.astype(jnp.int32)),
                    nbelow + jnp.sum(below.astype(jnp.int32)))

        n_own, nbelow = lax.fori_loop(0, nchunks, pass1,
                                      (jnp.int32(0), jnp.int32(0)))
        n_own = jnp.minimum(n_own, PCAP)

        for t in range((SLABC + 16) // 16):
            hist_v[pl.ds(16 * t, 16)] = zero

        def pass2(j, _):
            b = j * 16
            lc = ccid_v[pl.ds(b, 16)]
            lcs, _, rank, last, valid = _chunk_ranks(lc, n_own - b)
            lcc = jnp.minimum(lcs, SLABC - 1)
            plsc.addupdate_scatter(hist_v, [lcc], rank + 1, mask=last & valid)
            return 0

        nbch = (n_own + 15) >> 4
        lax.fori_loop(0, nbch, pass2, 0)

        carry = jnp.int32(0)
        for t in range(SLABC // 16):
            h = hist_v[pl.ds(16 * t, 16)]
            cs = plsc.cumsum(h)
            starts_v[pl.ds(16 * t, 16)] = cs - h + carry
            cursor_v[pl.ds(16 * t, 16)] = cs - h + carry
            carry = carry + jnp.sum(h)

        def pass3(j, _):
            b = j * 16
            lc = ccid_v[pl.ds(b, 16)]
            lcs, perm, rank, last, valid = _chunk_ranks(lc, n_own - b)
            lcc = jnp.minimum(lcs, SLABC - 1)
            sl = pl.ds(b, 16)
            xv = jnp.take(cx_v[sl], perm)
            yv = jnp.take(cy_v[sl], perm)
            zv = jnp.take(cz_v[sl], perm)
            iv = jnp.take(cidx_v[sl], perm)
            g = plsc.load_gather(cursor_v, [lcc])
            dst = jnp.minimum(g + rank, SCAP - 1)
            plsc.store_scatter(bx_v, [dst], xv, mask=valid)
            plsc.store_scatter(by_v, [dst], yv, mask=valid)
            plsc.store_scatter(bz_v, [dst], zv, mask=valid)
            plsc.store_scatter(bi_v, [dst], iv, mask=valid)
            plsc.store_scatter(cursor_v, [lcc], g + rank + 1,
                               mask=last & valid)
            return 0

        lax.fori_loop(0, nbch, pass3, 0)

        fill = pl.ds(jnp.minimum(n_own, SCAP), 16)
        two = jnp.full((16,), 2.0, jnp.float32)
        bx_v[fill] = two
        by_v[fill] = two
        bz_v[fill] = two
        bi_v[fill] = zero

        pltpu.sync_copy(bx_v.at[pl.ds(0, SCAP)], sx_hbm.at[pl.ds(wid * SCAP, SCAP)])
        pltpu.sync_copy(by_v.at[pl.ds(0, SCAP)], sy_hbm.at[pl.ds(wid * SCAP, SCAP)])
        pltpu.sync_copy(bz_v.at[pl.ds(0, SCAP)], sz_hbm.at[pl.ds(wid * SCAP, SCAP)])
        pltpu.sync_copy(bi_v.at[pl.ds(0, SCAP)], si_hbm.at[pl.ds(wid * SCAP, SCAP)])
        pltpu.sync_copy(starts_v.at[pl.ds(0, SLABC)],
                        csl_hbm.at[pl.ds(wid * SLABC, SLABC)])
        st_v[pl.ds(0, 16)] = jnp.where(iota == 0, n_own, 0)
        pltpu.sync_copy(st_v.at[pl.ds(0, 8)], stt_hbm.at[pl.ds(wid * 8, 8)])

    sx, sy, sz, si, csl, stt = bin_kernel(inp_flat)

    @functools.partial(
        pl.kernel,
        out_type=[
            jax.ShapeDtypeStruct((M * 64,), jnp.int32),
            jax.ShapeDtypeStruct((M * 64,), jnp.float32),
            jax.ShapeDtypeStruct((NW * QPW,), jnp.int32),
        ],
        mesh=mesh,
        compiler_params=cparams,
        scratch_types=[
            pltpu.VMEM((stage_cap,), jnp.float32),
            pltpu.VMEM((stage_cap,), jnp.float32),
            pltpu.VMEM((stage_cap,), jnp.float32),
            pltpu.VMEM((stage_cap,), jnp.int32),
            pltpu.VMEM((CPAD + 16,), jnp.int32),
            pltpu.VMEM((QPW * 3 + 16,), jnp.float32),
            pltpu.VMEM((NW * 8,), jnp.int32),
            pltpu.VMEM((NW + 16,), jnp.int32),
            pltpu.VMEM((KCAP + 16,), jnp.float32),
            pltpu.VMEM((KCAP + 16,), jnp.int32),
            pltpu.VMEM((1024,), jnp.int32),
            pltpu.VMEM((1024,), jnp.float32),
            pltpu.VMEM((QPW,), jnp.int32),
        ],
    )
    def query_kernel(sx_hbm, sy_hbm, sz_hbm, si_hbm, csl_hbm, stt_hbm, q_hbm,
                     oi_hbm, od_hbm, cnt_hbm,
                     sx_v, sy_v, sz_v, si_v, cs_v, q_v, sl_v, pb_v,
                     key_v, val_v, oi_v, od_v, cb_v):
        wid = _wid()
        iota = _iota()
        infv = jnp.full((16,), jnp.inf, jnp.float32)
        r2v = jnp.full((16,), R2F, jnp.float32)

        pltpu.sync_copy(stt_hbm, sl_v)
        t_lo = plsc.load_gather(sl_v, [iota * 8])
        t_hi = plsc.load_gather(sl_v, [(iota + 16) * 8])
        p_lo = jnp.bitwise_and(t_lo + 7, -8)
        p_hi = jnp.bitwise_and(t_hi + 7, -8)
        c_lo = plsc.cumsum(p_lo)
        c_hi = plsc.cumsum(p_hi) + c_lo[15]
        pb_v[pl.ds(0, 16)] = c_lo - p_lo
        pb_v[pl.ds(16, 16)] = c_hi - p_hi

        pltpu.sync_copy(csl_hbm, cs_v.at[pl.ds(0, CPAD)])

        def rebase(s, _):
            base = pb_v[pl.ds(s, 16)][0]
            for t in range(SLABC // 16):
                sl = pl.ds(s * SLABC + 16 * t, 16)
                cs_v[sl] = cs_v[sl] + base
            return 0

        lax.fori_loop(0, NW, rebase, 0)

        def stage(s, _):
            base = pl.multiple_of(pb_v[pl.ds(s, 16)][0], 8)
            src = pl.ds(s * SCAP, SCAP)
            dst = pl.ds(base, SCAP)
            pltpu.sync_copy(sx_hbm.at[src], sx_v.at[dst])
            pltpu.sync_copy(sy_hbm.at[src], sy_v.at[dst])
            pltpu.sync_copy(sz_hbm.at[src], sz_v.at[dst])
            pltpu.sync_copy(si_hbm.at[src], si_v.at[dst])
            return 0

        lax.fori_loop(0, NW, stage, 0)

        qlo = wid * QPW
        pltpu.sync_copy(q_hbm.at[pl.ds(qlo * 3, QPW * 3)],
                        q_v.at[pl.ds(0, QPW * 3)])
        ngroups = jnp.minimum(M - qlo, QPW) >> 4

        def group(gi, _):
            def one_query(j, countv):
                qoff = (gi * 16 + j) * 3
                qv = q_v[pl.ds(qoff, 16)]
                qxv = jnp.full((16,), qv[0], jnp.float32)
                qyv = jnp.full((16,), qv[1], jnp.float32)
                qzv = jnp.full((16,), qv[2], jnp.float32)
                bxv = _bf16_rne(qxv)
                byv = _bf16_rne(qyv)
                bzv = _bf16_rne(qzv)
                qsqv = (qxv * qxv + qzv * qzv) + qyv * qyv
                bqv = _bf16_rne(qv)
                bsqv = (bxv * bxv + byv * byv) + bzv * bzv
                rho = jnp.maximum(
                    r2v - (qsqv - bsqv) + (qsqv + 0.8) * (1.0 / 256.0)
                    + 0.001, 0.0)
                reach = rho * _isqrt(rho) + 0.006
                lov = jnp.clip(((bqv - reach) * 20.0).astype(jnp.int32),
                               0, G - 1)
                hiv = jnp.clip(((bqv + reach) * 20.0).astype(jnp.int32),
                               0, G - 1)
                x0, y0, z0 = lov[0], lov[1], lov[2]
                x1, y1, z1 = hiv[0], hiv[1], hiv[2]
                for t in range(5):
                    key_v[pl.ds(16 * t, 16)] = infv

                def xloop(cx, off):
                    def yloop(cy, off):
                        c0 = (cx * G + cy) * G + z0
                        c1e = (cx * G + cy) * G + z1 + 1
                        b = cs_v[pl.ds(c0, 16)][0]
                        e = cs_v[pl.ds(c1e, 16)][0]

                        def scan_body(c):
                            bb, ee, oo = c
                            px = sx_v[pl.ds(bb, 16)]
                            py = sy_v[pl.ds(bb, 16)]
                            pz = sz_v[pl.ds(bb, 16)]
                            piv = si_v[pl.ds(bb, 16)]
                            bpx = _bf16_rne(px)
                            bpy = _bf16_rne(py)
                            bpz = _bf16_rne(pz)
                            dot = _exact_sum3(bxv * bpx, byv * bpy,
                                              bzv * bpz)
                            psq = (px * px + pz * pz) + py * py
                            d2 = (qsqv + psq) - 2.0 * dot
                            d2 = jnp.maximum(d2, 0.0)
                            keep = (d2 <= r2v) & ((bb + iota) < ee)
                            keyv = jnp.where(
                                d2 == 0.0,
                                piv.astype(jnp.float32) * 1e-20, d2)
                            oc = jnp.minimum(oo, KCAP)
                            plsc.store_compressed(key_v.at[pl.ds(oc, 16)],
                                                  keyv, mask=keep)
                            plsc.store_compressed(val_v.at[pl.ds(oc, 16)],
                                                  piv, mask=keep)
                            return (bb + 16, ee,
                                    oo + jnp.sum(keep.astype(jnp.int32)))

                        _, _, off = lax.while_loop(
                            lambda c: c[0] < c[1], scan_body, (b, e, off))
                        return off

                    return lax.fori_loop(y0, y1 + 1, yloop, off)

                off = lax.fori_loop(x0, x1 + 1, xloop, jnp.int32(0))
                off = jnp.minimum(off, KCAP)
                key_v[pl.ds(off, 16)] = infv
                k0 = key_v[pl.ds(0, 16)]
                k1 = key_v[pl.ds(16, 16)]
                k2 = key_v[pl.ds(32, 16)]
                k3 = key_v[pl.ds(48, 16)]
                v0 = val_v[pl.ds(0, 16)]
                v1 = val_v[pl.ds(16, 16)]
                v2 = val_v[pl.ds(32, 16)]
                v3 = val_v[pl.ds(48, 16)]
                R = _sort64(k0, v0, k1, v1, k2, v2, k3, v3)

                def absorb(c, R):
                    ck = key_v[pl.ds(c * 16, 16)]
                    cv = val_v[pl.ds(c * 16, 16)]
                    ck, cv = _sortv(ck, cv)
                    rk = lax.rev(ck, (0,))
                    rv = lax.rev(cv, (0,))
                    m = R[6] <= rk
                    nk = jnp.where(m, R[6], rk)
                    nv = jnp.where(m, R[7], rv)
                    return _sort64(R[0], R[1], R[2], R[3], R[4], R[5], nk, nv)

                nch = (off + 15) >> 4
                R = lax.fori_loop(4, jnp.maximum(nch, 4), absorb, R)
                R = _tie_fix(R, 0)
                R = _tie_fix(R, 1)
                R = _tie_fix(R, 0)

                obase = j * 64
                for t in range(4):
                    kk = R[2 * t]
                    vv = R[2 * t + 1]
                    valid = kk <= r2v
                    dist = jnp.where(valid & (kk > 1e-10),
                                     kk * _isqrt(kk), 0.0)
                    oi_v[pl.ds(obase + 16 * t, 16)] = jnp.where(valid, vv, -1)
                    od_v[pl.ds(obase + 16 * t, 16)] = dist
                cnt = jnp.minimum(off, 64)
                return jnp.where(iota == j, cnt, countv)

            countv = lax.fori_loop(0, 16, one_query, jnp.zeros((16,), jnp.int32))
            qbase = qlo + gi * 16
            pltpu.sync_copy(oi_v, oi_hbm.at[pl.ds(qbase * 64, 1024)])
            pltpu.sync_copy(od_v, od_hbm.at[pl.ds(qbase * 64, 1024)])
            cb_v[pl.ds(gi * 16, 16)] = countv
            return 0

        lax.fori_loop(0, ngroups, group, 0)
        pltpu.sync_copy(cb_v, cnt_hbm.at[pl.ds(qlo, QPW)])

    oi, od, counts = query_kernel(sx, sy, sz, si, csl, stt, q_flat)

    RSP = (M + 1 + 15) // 16 * 16

    @functools.partial(
        pl.kernel,
        out_type=jax.ShapeDtypeStruct((RSP,), jnp.int32),
        mesh=mesh,
        compiler_params=cparams,
        scratch_types=[
            pltpu.VMEM((NW * QPW,), jnp.int32),
            pltpu.VMEM((RSP,), jnp.int32),
        ],
    )
    def scan_kernel(cnt_hbm, rs_hbm, cv_v, rs_v):
        wid = _wid()

        @pl.when(wid == 0)
        def _():
            pltpu.sync_copy(cnt_hbm, cv_v)
            zero = jnp.zeros((16,), jnp.int32)
            rs_v[pl.ds(0, 16)] = zero
            rs_v[pl.ds(RSP - 16, 16)] = zero

            def body(i, carry):
                v = cv_v[pl.ds(16 * i, 16)]
                s = plsc.cumsum(v) + carry
                rs_v[pl.ds(1 + 16 * i, 16)] = s
                return s[15]

            lax.fori_loop(0, M // 16, body, jnp.int32(0))
            pltpu.sync_copy(rs_v, rs_hbm)

    rs = scan_kernel(counts)

    neighbors_index = oi.reshape(M, 64)
    neighbors_distance = od.reshape(M, 64)
    neighbors_row_splits = rs[:M + 1]
    return neighbors_index, neighbors_row_splits, neighbors_distance
